# Initial kernel scaffold; baseline (speedup 1.0000x reference)
import jax
import jax.numpy as jnp
from jax.experimental import pallas as pl


def kernel(x, router_w, router_b, w1, b1, w2, b2):
    raise NotImplementedError("write your pallas kernel here")


# speedup vs baseline: 1.9119x; 1.9119x over previous
import functools
import math

import jax
import jax.numpy as jnp
from jax import lax
from jax.experimental import pallas as pl
from jax.experimental.pallas import tpu as pltpu
from jax.experimental.pallas import tpu_sc as plsc

EMB = 1024
HID = 1536
NEXP = 8
TOPK = 2
BLK = 128
PADCAP = 9216
NBLK = PADCAP // BLK


def _router_body(x_ref, rw_ref, rb_ref, e_ref, w_ref):
    xb = x_ref[...]
    logits = jnp.dot(xb, rw_ref[...], preferred_element_type=jnp.float32)
    logits = logits + rb_ref[...]
    col = lax.broadcasted_iota(jnp.int32, logits.shape, 1)
    valid = col < NEXP
    neg = jnp.float32(-1e30)
    lg = jnp.where(valid, logits, neg)
    m1 = jnp.max(lg, axis=1, keepdims=True)
    a1 = jnp.min(jnp.where(lg == m1, col, 1000000), axis=1, keepdims=True)
    sel1 = col == a1
    lg2 = jnp.where(sel1, neg, lg)
    m2 = jnp.max(lg2, axis=1, keepdims=True)
    a2 = jnp.min(jnp.where(lg2 == m2, col, 1000000), axis=1, keepdims=True)
    sel2 = col == a2
    ex = jnp.where(valid, jnp.exp(lg - m1), 0.0)
    denom = jnp.sum(ex, axis=1, keepdims=True)
    p = ex / denom
    w1v = jnp.sum(jnp.where(sel1, p, 0.0), axis=1, keepdims=True)
    w2v = jnp.sum(jnp.where(sel2, p, 0.0), axis=1, keepdims=True)
    e_ref[...] = jnp.where(col == 0, a1, jnp.where(col == 1, a2, 0)).astype(
        jnp.int32)
    w_ref[...] = jnp.where(col == 0, w1v, jnp.where(col == 1, w2v, 0.0))


def _router(xf, router_w, router_b):
    t = xf.shape[0]
    rows = 512
    rw_p = jnp.zeros((EMB, 128), jnp.float32).at[:, :NEXP].set(router_w)
    rb_p = jnp.zeros((1, 128), jnp.float32).at[0, :NEXP].set(router_b)
    e_full, w_full = pl.pallas_call(
        _router_body,
        grid=(t // rows,),
        in_specs=[
            pl.BlockSpec((rows, EMB), lambda i: (i, 0)),
            pl.BlockSpec((EMB, 128), lambda i: (0, 0)),
            pl.BlockSpec((1, 128), lambda i: (0, 0)),
        ],
        out_specs=[
            pl.BlockSpec((rows, 128), lambda i: (i, 0)),
            pl.BlockSpec((rows, 128), lambda i: (i, 0)),
        ],
        out_shape=[
            jax.ShapeDtypeStruct((t, 128), jnp.int32),
            jax.ShapeDtypeStruct((t, 128), jnp.float32),
        ],
    )(xf, rw_p, rb_p)
    return e_full[:, :TOPK], w_full[:, :TOPK]


def _dispatch(e2, w2f):
    t = e2.shape[0]
    s = t * TOPK
    ef = e2.reshape(s).astype(jnp.int32)
    wf = w2f.reshape(s)
    tokf = jnp.arange(s, dtype=jnp.int32) // TOPK
    onehot = (ef[:, None] == jnp.arange(NEXP, dtype=jnp.int32)[None, :])
    onehot = onehot.astype(jnp.int32)
    counts = jnp.sum(onehot, axis=0)
    padded = ((counts + BLK - 1) // BLK) * BLK
    off = jnp.concatenate(
        [jnp.zeros((1,), jnp.int32), jnp.cumsum(padded)])[:NEXP]
    rank = jnp.cumsum(onehot, axis=0) - onehot
    rk = jnp.take_along_axis(rank, ef[:, None], axis=1)[:, 0]
    pos = (off[ef] + rk).astype(jnp.int32)
    sorted_tok = jnp.zeros((PADCAP,), jnp.int32).at[pos].set(tokf)
    wsorted = jnp.zeros((PADCAP,), jnp.float32).at[pos].set(wf)
    starts = jnp.arange(NBLK, dtype=jnp.int32) * BLK
    be = jnp.sum((starts[:, None] >= off[None, 1:]).astype(jnp.int32), axis=1)
    ws2d = jnp.broadcast_to(wsorted[:, None], (PADCAP, 128))
    return sorted_tok, be, pos, ws2d


def _sc_gather(table, idx, chunk):
    n, d = idx.shape[0], table.shape[1]
    info = plsc.get_sparse_core_info()
    nw = info.num_cores * info.num_subcores
    per_w = n // nw
    n_chunks = per_w // chunk
    mesh = plsc.VectorSubcoreMesh(core_axis_name="c", subcore_axis_name="s")

    @functools.partial(
        pl.kernel,
        mesh=mesh,
        out_type=jax.ShapeDtypeStruct((n, d), jnp.float32),
        scratch_types=[
            pltpu.VMEM((chunk,), jnp.int32),
            pltpu.VMEM((chunk, d), jnp.float32),
            pltpu.SemaphoreType.DMA,
        ],
    )
    def gk(tab_hbm, idx_hbm, out_hbm, idx_v, rows_v, sem):
        wid = lax.axis_index("s") * info.num_cores + lax.axis_index("c")
        base = wid * per_w
        for c in range(n_chunks):
            o = base + c * chunk
            pltpu.sync_copy(idx_hbm.at[pl.ds(o, chunk)], idx_v)
            pltpu.async_copy(tab_hbm.at[idx_v], rows_v, sem).wait()
            pltpu.sync_copy(rows_v, out_hbm.at[pl.ds(o, chunk)])

    return gk(table, idx)


def _ffn_body(be_ref, xs_ref, w1_ref, b1_ref, w2_ref, b2_ref, ws_ref, y_ref):
    del be_ref
    xb = xs_ref[...]
    h = jnp.dot(xb, w1_ref[0], preferred_element_type=jnp.float32)
    h = h + b1_ref[0]
    h = 0.5 * h * (1.0 + lax.erf(h * jnp.float32(1.0 / math.sqrt(2.0))))
    y = jnp.dot(h, w2_ref[0], preferred_element_type=jnp.float32)
    y = y + b2_ref[0]
    y_ref[...] = y * ws_ref[:, 0:1]


def _ffn(block_expert, xs, w1, b1, w2, b2, ws2d):
    grid_spec = pltpu.PrefetchScalarGridSpec(
        num_scalar_prefetch=1,
        grid=(NBLK,),
        in_specs=[
            pl.BlockSpec((BLK, EMB), lambda i, be: (i, 0)),
            pl.BlockSpec((1, EMB, HID), lambda i, be: (be[i], 0, 0)),
            pl.BlockSpec((1, 1, HID), lambda i, be: (be[i], 0, 0)),
            pl.BlockSpec((1, HID, EMB), lambda i, be: (be[i], 0, 0)),
            pl.BlockSpec((1, 1, EMB), lambda i, be: (be[i], 0, 0)),
            pl.BlockSpec((BLK, 128), lambda i, be: (i, 0)),
        ],
        out_specs=pl.BlockSpec((BLK, EMB), lambda i, be: (i, 0)),
    )
    return pl.pallas_call(
        _ffn_body,
        grid_spec=grid_spec,
        out_shape=jax.ShapeDtypeStruct((PADCAP, EMB), jnp.float32),
    )(block_expert, xs, w1, b1[:, None, :], w2, b2[:, None, :], ws2d)


def _sc_combine(y, pos, t):
    d = y.shape[1]
    info = plsc.get_sparse_core_info()
    nw = info.num_cores * info.num_subcores
    per_w = t // nw
    sub = 32
    n_sub = per_w // sub
    mesh = plsc.VectorSubcoreMesh(core_axis_name="c", subcore_axis_name="s")

    @functools.partial(
        pl.kernel,
        mesh=mesh,
        out_type=jax.ShapeDtypeStruct((t, d), jnp.float32),
        scratch_types=[
            pltpu.VMEM((sub * 2,), jnp.int32),
            pltpu.VMEM((sub * 2, d), jnp.float32),
            pltpu.VMEM((sub, d), jnp.float32),
            pltpu.SemaphoreType.DMA,
        ],
    )
    def ck(y_hbm, pos_hbm, out_hbm, idx_s, rows_v, out_v, sem):
        wid = lax.axis_index("s") * info.num_cores + lax.axis_index("c")
        for sc in range(n_sub):
            pltpu.sync_copy(
                pos_hbm.at[pl.ds(wid * per_w * 2 + sc * sub * 2, sub * 2)],
                idx_s)
            pltpu.async_copy(y_hbm.at[idx_s], rows_v, sem).wait()

            def body(j, carry):
                for dd in range(d // 16):
                    sl = pl.ds(dd * 16, 16)
                    out_v[j, sl] = rows_v[2 * j, sl] + rows_v[2 * j + 1, sl]
                return carry

            lax.fori_loop(0, sub, body, 0)
            pltpu.sync_copy(out_v,
                            out_hbm.at[pl.ds(wid * per_w + sc * sub, sub)])

    return ck(y, pos)


def kernel(x, router_w, router_b, w1, b1, w2, b2):
    bv, nv, ev = x.shape
    t = bv * nv
    xf = x.reshape(t, ev)
    e2, wpair = _router(xf, router_w, router_b)
    sorted_tok, block_expert, pos, ws2d = _dispatch(e2, wpair)
    xs = _sc_gather(xf, sorted_tok, chunk=96)
    y = _ffn(block_expert, xs, w1, b1, w2, b2, ws2d)
    out = _sc_combine(y, pos, t)
    return out.reshape(bv, nv, ev)
